# Initial kernel scaffold; baseline (speedup 1.0000x reference)
import functools

import jax
import jax.numpy as jnp
from jax import lax
from jax.experimental import pallas as pl
from jax.experimental.pallas import tpu as pltpu
from jax.experimental.pallas import tpu_sc as plsc

N_NODES = 10000
N_EDGES = 320000
D = 128

NC, NS = 2, 16
NW = NC * NS
E_SHARD = N_EDGES // NW
CH = 80
NCHUNK = E_SHARD // CH
ROWS_T = N_NODES // NS
DEGW = 16

_MESH = plsc.VectorSubcoreMesh(
    core_axis_name="c", subcore_axis_name="s", num_cores=NC, num_subcores=NS)


@functools.partial(
    pl.kernel,
    out_type=jax.ShapeDtypeStruct((NC, N_NODES, DEGW), jnp.float32),
    mesh=_MESH,
    scratch_types=[
        pltpu.VMEM((CH,), jnp.int32),
        pltpu.VMEM((CH, DEGW), jnp.float32),
        pltpu.VMEM((ROWS_T, DEGW), jnp.float32),
        pltpu.VMEM_SHARED((N_NODES, DEGW), jnp.float32),
        pltpu.SemaphoreType.DMA,
    ],
)
def _sc_degree(dst_hbm, ones_hbm, zeros_hbm, out_hbm, dstv, onesv, zerov,
               hist, sem):
    c = lax.axis_index("c")
    s = lax.axis_index("s")
    shard_base = (s * NC + c) * E_SHARD
    row0 = s * ROWS_T
    pltpu.sync_copy(zeros_hbm, zerov)
    pltpu.sync_copy(zerov, hist.at[pl.ds(row0, ROWS_T)])
    pltpu.sync_copy(ones_hbm, onesv)
    plsc.subcore_barrier()

    def body(j, carry):
        pltpu.sync_copy(dst_hbm.at[pl.ds(shard_base + j * CH, CH)], dstv)
        pltpu.sync_copy(onesv, hist.at[dstv], add=True)
        return carry

    lax.fori_loop(0, NCHUNK, body, 0)
    plsc.subcore_barrier()
    pltpu.sync_copy(hist.at[pl.ds(row0, ROWS_T)],
                    out_hbm.at[c, pl.ds(row0, ROWS_T)])


@functools.partial(
    pl.kernel,
    out_type=jax.ShapeDtypeStruct((NC, N_NODES, D), jnp.float32),
    mesh=_MESH,
    scratch_types=[
        pltpu.VMEM((CH,), jnp.int32),
        pltpu.VMEM((CH,), jnp.int32),
        pltpu.VMEM((CH, D), jnp.float32),
        pltpu.VMEM((ROWS_T // 5, D), jnp.float32),
        pltpu.VMEM_SHARED((N_NODES, D), jnp.float32),
        pltpu.SemaphoreType.DMA,
    ],
)
def _sc_aggregate(y_hbm, src_hbm, dst_hbm, zeros_hbm, out_hbm, srcv, dstv,
                  rows, zrow, acc, sem):
    c = lax.axis_index("c")
    s = lax.axis_index("s")
    shard_base = (s * NC + c) * E_SHARD
    row0 = s * ROWS_T
    zr = ROWS_T // 5
    pltpu.sync_copy(zeros_hbm, zrow)
    for k in range(5):
        pltpu.sync_copy(zrow, acc.at[pl.ds(row0 + k * zr, zr)])
    plsc.subcore_barrier()

    def body(j, carry):
        b = shard_base + j * CH
        pltpu.sync_copy(src_hbm.at[pl.ds(b, CH)], srcv)
        pltpu.sync_copy(dst_hbm.at[pl.ds(b, CH)], dstv)
        pltpu.async_copy(y_hbm.at[srcv], rows, sem).wait()
        pltpu.sync_copy(rows, acc.at[dstv], add=True)
        return carry

    lax.fori_loop(0, NCHUNK, body, 0)
    plsc.subcore_barrier()
    pltpu.sync_copy(acc.at[pl.ds(row0, ROWS_T)],
                    out_hbm.at[c, pl.ds(row0, ROWS_T)])


_BR = 400
_GRID = N_NODES // _BR


def _dis_from_degp(degp):
    deg = degp[0, :, 0] + degp[1, :, 0] + 1.0
    return lax.rsqrt(deg)


def _tc_y1_body(x_ref, w_ref, degp_ref, o_ref):
    dis = _dis_from_degp(degp_ref[...])
    xw = jnp.dot(x_ref[...], w_ref[...], preferred_element_type=jnp.float32)
    o_ref[...] = xw * dis[:, None]


def _tc_mid_body(aggp_ref, y_ref, degp_ref, b_ref, w_ref, o_ref):
    dis = _dis_from_degp(degp_ref[...])
    aggp = aggp_ref[...]
    h = jax.nn.relu(dis[:, None] * (aggp[0] + aggp[1] + y_ref[...])
                    + b_ref[...])
    o_ref[...] = jnp.dot(h, w_ref[...],
                         preferred_element_type=jnp.float32) * dis[:, None]


def _tc_out_body(aggp_ref, y_ref, degp_ref, b_ref, o_ref):
    dis = _dis_from_degp(degp_ref[...])
    aggp = aggp_ref[...]
    o_ref[...] = jax.nn.relu(dis[:, None] * (aggp[0] + aggp[1] + y_ref[...])
                             + b_ref[...])


_ROWB = pl.BlockSpec((_BR, D), lambda i: (i, 0))
_AGGB = pl.BlockSpec((NC, _BR, D), lambda i: (0, i, 0))
_DEGB = pl.BlockSpec((NC, _BR, DEGW), lambda i: (0, i, 0))
_WB = pl.BlockSpec((D, D), lambda i: (0, 0))
_BB = pl.BlockSpec((1, D), lambda i: (0, 0))
_OUT = jax.ShapeDtypeStruct((N_NODES, D), jnp.float32)


def _tc_y1(x, w1, degp):
    return pl.pallas_call(
        _tc_y1_body, grid=(_GRID,),
        in_specs=[_ROWB, _WB, _DEGB], out_specs=_ROWB, out_shape=_OUT,
    )(x, w1, degp)


def _tc_mid(aggp, y, degp, b, w2):
    return pl.pallas_call(
        _tc_mid_body, grid=(_GRID,),
        in_specs=[_AGGB, _ROWB, _DEGB, _BB, _WB], out_specs=_ROWB,
        out_shape=_OUT,
    )(aggp, y, degp, b, w2)


def _tc_out(aggp, y, degp, b):
    return pl.pallas_call(
        _tc_out_body, grid=(_GRID,),
        in_specs=[_AGGB, _ROWB, _DEGB, _BB], out_specs=_ROWB, out_shape=_OUT,
    )(aggp, y, degp, b)


def kernel(x, edge_index, W1, b1, W2, b2):
    ei = edge_index.astype(jnp.int32)
    src, dst = ei[0], ei[1]
    ones16 = jnp.ones((CH, DEGW), jnp.float32)
    zeros16 = jnp.zeros((ROWS_T, DEGW), jnp.float32)
    zeros128 = jnp.zeros((ROWS_T // 5, D), jnp.float32)
    b1r = b1.reshape(1, D)
    b2r = b2.reshape(1, D)

    degp = _sc_degree(dst, ones16, zeros16)
    y1 = _tc_y1(x, W1, degp)
    aggp1 = _sc_aggregate(y1, src, dst, zeros128)
    y2 = _tc_mid(aggp1, y1, degp, b1r, W2)
    aggp2 = _sc_aggregate(y2, src, dst, zeros128)
    return _tc_out(aggp2, y2, degp, b2r)


# speedup vs baseline: 12.5523x; 12.5523x over previous
import functools

import jax
import jax.numpy as jnp
from jax import lax
from jax.experimental import pallas as pl
from jax.experimental.pallas import tpu as pltpu
from jax.experimental.pallas import tpu_sc as plsc

N_NODES = 10000
N_PAD = 10240
N_EDGES = 320000
D = 128

NC, NS = 2, 16
NW = NC * NS
E_SHARD = N_EDGES // NW
CH = 80
NCHUNK = E_SHARD // CH
ROWS_T = N_PAD // NS
ZR = 128

_MESH = plsc.VectorSubcoreMesh(
    core_axis_name="c", subcore_axis_name="s", num_cores=NC, num_subcores=NS)


@functools.partial(
    pl.kernel,
    out_type=jax.ShapeDtypeStruct((NC * N_PAD,), jnp.float32),
    mesh=_MESH,
    scratch_types=[
        pltpu.VMEM((CH,), jnp.int32),
        pltpu.VMEM((CH,), jnp.float32),
        pltpu.VMEM((ROWS_T,), jnp.float32),
        pltpu.VMEM_SHARED((N_PAD,), jnp.float32),
        pltpu.SemaphoreType.DMA,
    ],
)
def _sc_degree(dst_hbm, out_hbm, dstv, onesv, zerov, hist, sem):
    c = lax.axis_index("c")
    s = lax.axis_index("s")
    shard_base = (s * NC + c) * E_SHARD
    row0 = s * ROWS_T
    for k in range(CH // 16):
        onesv[pl.ds(k * 16, 16)] = jnp.ones((16,), jnp.float32)
    for k in range(ROWS_T // 16):
        zerov[pl.ds(k * 16, 16)] = jnp.zeros((16,), jnp.float32)
    pltpu.sync_copy(zerov, hist.at[pl.ds(row0, ROWS_T)])
    plsc.subcore_barrier()

    def body(j, carry):
        pltpu.sync_copy(dst_hbm.at[pl.ds(shard_base + j * CH, CH)], dstv)
        pltpu.sync_copy(onesv, hist.at[dstv], add=True)
        return carry

    lax.fori_loop(0, NCHUNK, body, 0)
    plsc.subcore_barrier()
    pltpu.sync_copy(hist.at[pl.ds(row0, ROWS_T)], zerov)
    pltpu.sync_copy(zerov, out_hbm.at[pl.ds(c * N_PAD + row0, ROWS_T)])


@functools.partial(
    pl.kernel,
    out_type=jax.ShapeDtypeStruct((NC * N_PAD, D), jnp.float32),
    mesh=_MESH,
    scratch_types=[
        pltpu.VMEM((CH,), jnp.int32),
        pltpu.VMEM((CH,), jnp.int32),
        pltpu.VMEM((CH, D), jnp.float32),
        pltpu.VMEM((ZR, D), jnp.float32),
        pltpu.VMEM_SHARED((N_PAD, D), jnp.float32),
        pltpu.SemaphoreType.DMA,
    ],
)
def _sc_aggregate(y_hbm, src_hbm, dst_hbm, zeros_hbm, out_hbm, srcv, dstv,
                  rows, zrow, acc, sem):
    c = lax.axis_index("c")
    s = lax.axis_index("s")
    shard_base = (s * NC + c) * E_SHARD
    row0 = s * ROWS_T
    pltpu.sync_copy(zeros_hbm, zrow)
    for k in range(ROWS_T // ZR):
        pltpu.sync_copy(zrow, acc.at[pl.ds(row0 + k * ZR, ZR)])
    plsc.subcore_barrier()

    def body(j, carry):
        b = shard_base + j * CH
        pltpu.sync_copy(src_hbm.at[pl.ds(b, CH)], srcv)
        pltpu.sync_copy(dst_hbm.at[pl.ds(b, CH)], dstv)
        pltpu.async_copy(y_hbm.at[srcv], rows, sem).wait()
        pltpu.sync_copy(rows, acc.at[dstv], add=True)
        return carry

    lax.fori_loop(0, NCHUNK, body, 0)
    plsc.subcore_barrier()
    for k in range(ROWS_T // ZR):
        pltpu.sync_copy(acc.at[pl.ds(row0 + k * ZR, ZR)], zrow)
        pltpu.sync_copy(
            zrow, out_hbm.at[pl.ds(c * N_PAD + row0 + k * ZR, ZR)])


_BR = 400
_GRID = N_NODES // _BR


def _dis(degp_ref):
    seg = degp_ref[...]
    return lax.rsqrt(seg[:, 0] + seg[:, 1] + 1.0)


def _tc_y1_body(x_ref, w_ref, degp_ref, o_ref):
    dis = _dis(degp_ref)
    xw = jnp.dot(x_ref[...], w_ref[...], preferred_element_type=jnp.float32)
    o_ref[...] = xw * dis[:, None]


def _tc_mid_body(a0_ref, a1_ref, y_ref, degp_ref, b_ref, w_ref, o_ref):
    dis = _dis(degp_ref)
    h = jax.nn.relu(dis[:, None] * (a0_ref[...] + a1_ref[...] + y_ref[...])
                    + b_ref[...])
    o_ref[...] = jnp.dot(h, w_ref[...],
                         preferred_element_type=jnp.float32) * dis[:, None]


def _tc_out_body(a0_ref, a1_ref, y_ref, degp_ref, b_ref, o_ref):
    dis = _dis(degp_ref)
    o_ref[...] = jax.nn.relu(
        dis[:, None] * (a0_ref[...] + a1_ref[...] + y_ref[...]) + b_ref[...])


_ROWB = pl.BlockSpec((_BR, D), lambda i: (i, 0))
_DEGB = pl.BlockSpec((_BR, NC), lambda i: (i, 0))
_WB = pl.BlockSpec((D, D), lambda i: (0, 0))
_BB = pl.BlockSpec((1, D), lambda i: (0, 0))
_OUT = jax.ShapeDtypeStruct((N_NODES, D), jnp.float32)


def _tc_y1(x, w1, degp):
    return pl.pallas_call(
        _tc_y1_body, grid=(_GRID,),
        in_specs=[_ROWB, _WB, _DEGB], out_specs=_ROWB, out_shape=_OUT,
    )(x, w1, degp)


def _tc_mid(a0, a1, y, degp, b, w2):
    return pl.pallas_call(
        _tc_mid_body, grid=(_GRID,),
        in_specs=[_ROWB, _ROWB, _ROWB, _DEGB, _BB, _WB], out_specs=_ROWB,
        out_shape=_OUT,
    )(a0, a1, y, degp, b, w2)


def _tc_out(a0, a1, y, degp, b):
    return pl.pallas_call(
        _tc_out_body, grid=(_GRID,),
        in_specs=[_ROWB, _ROWB, _ROWB, _DEGB, _BB], out_specs=_ROWB,
        out_shape=_OUT,
    )(a0, a1, y, degp, b)


def kernel(x, edge_index, W1, b1, W2, b2):
    ei = edge_index.astype(jnp.int32)
    src, dst = ei[0], ei[1]
    zeros128 = jnp.zeros((ZR, D), jnp.float32)
    b1r = b1.reshape(1, D)
    b2r = b2.reshape(1, D)

    degp = _sc_degree(dst).reshape(NC, N_PAD).T
    y1 = _tc_y1(x, W1, degp)

    aggp1 = _sc_aggregate(y1, src, dst, zeros128)
    a10 = aggp1[:N_PAD]
    a11 = aggp1[N_PAD:]
    y2 = _tc_mid(a10, a11, y1, degp, b1r, W2)

    aggp2 = _sc_aggregate(y2, src, dst, zeros128)
    a20 = aggp2[:N_PAD]
    a21 = aggp2[N_PAD:]
    return _tc_out(a20, a21, y2, degp, b2r)
